# baseline (speedup 1.0000x reference)
import functools

import jax
import jax.numpy as jnp
from jax import lax
from jax.experimental import pallas as pl
from jax.experimental.pallas import tpu as pltpu
from jax.experimental.pallas import tpu_sc as plsc

NC = 2
NS = 16
NW = NC * NS
L = 16

V = 1000
N_ROWS = 51200
B_PER_W = N_ROWS // NW
NG = V // 8
NT0 = N_ROWS // 128
TCH = 8
NCHUNK = NT0 // TCH
NPAIR = NCHUNK // 2


def _lse_body(tab_ref, out_ref):
    x = tab_ref[...]
    m = jnp.max(x, axis=1, keepdims=True)
    s = jnp.sum(jnp.exp(x - m), axis=1, keepdims=True)
    out_ref[...] = m + jnp.log(s)


def _row_lse(table):
    out = pl.pallas_call(
        _lse_body,
        out_shape=jax.ShapeDtypeStruct((V, 1), jnp.float32),
    )(table)
    return out.reshape(V)


VF = V * V + 8


def _sc_body(tabT_hbm, tabTf_hbm, idx_hbm, tgt_hbm, lse_hbm,
             out_hbm, part_hbm,
             idx_v, tgt_v, lse_v, tabT_v, buf0, buf1,
             ilist_v, ebuf_v, part_v,
             s0, s1, esem):
    c = lax.axis_index("c")
    s = lax.axis_index("s")
    wid = s * NC + c
    base = wid * B_PER_W

    g_start = wid * NG // NW
    g_end = (wid + 1) * NG // NW

    pltpu.sync_copy(tabT_hbm.at[pl.ds(g_start * 8, 32)], tabT_v)
    pltpu.sync_copy(idx_hbm, idx_v)
    pltpu.sync_copy(tgt_hbm.at[pl.ds(base, B_PER_W)], tgt_v)
    pltpu.sync_copy(lse_hbm, lse_v)

    def fill(buf, g, chunk):
        rows = [tabT_v.at[(g - g_start) * 8 + c_loc] for c_loc in range(8)]

        @plsc.parallel_loop(0, TCH, step=1, unroll=4)
        def _(t):
            r0 = (chunk * TCH + t) * 128
            idx16 = [idx_v[pl.ds(r0 + jj * L, L)] for jj in range(8)]
            for c_loc in range(8):
                for jj in range(8):
                    buf[t, c_loc, pl.ds(jj * L, L)] = plsc.load_gather(
                        rows[c_loc], [idx16[jj]])

    def scatter_start(g, chunk, buf, sem):
        pltpu.async_copy(
            buf, out_hbm.at[g, pl.ds(chunk * TCH, TCH)], sem)

    def scatter_wait(g, chunk, buf, sem):
        pltpu.make_async_copy(
            buf, out_hbm.at[g, pl.ds(chunk * TCH, TCH)], sem).wait()

    nq = (g_end - g_start) * NPAIR

    def q_body(q, carry):
        g = g_start + q // NPAIR
        p = q % NPAIR
        c0 = 2 * p
        c1 = 2 * p + 1

        @pl.when(q > 0)
        def _():
            scatter_wait(g, c0, buf0, s0)

        fill(buf0, g, c0)
        scatter_start(g, c0, buf0, s0)

        @pl.when(q > 0)
        def _():
            scatter_wait(g, c1, buf1, s1)

        fill(buf1, g, c1)
        scatter_start(g, c1, buf1, s1)
        return carry

    lax.fori_loop(0, nq, q_body, jnp.int32(0))

    for j in range(B_PER_W // L):
        i16 = idx_v[pl.ds(base + j * L, L)]
        t16 = tgt_v[pl.ds(j * L, L)]
        ilist_v[pl.ds(j * L, L)] = t16 * V + i16

    EG = 80
    for k in range(B_PER_W // EG):
        pltpu.async_copy(
            tabTf_hbm.at[ilist_v.at[pl.ds(k * EG, EG)]],
            ebuf_v.at[pl.ds(k * EG, EG)], esem)
    for k in range(B_PER_W // EG):
        pltpu.make_async_copy(
            tabTf_hbm.at[ilist_v.at[pl.ds(k * EG, EG)]],
            ebuf_v.at[pl.ds(k * EG, EG)], esem).wait()

    acc = jnp.zeros((L,), jnp.float32)
    for j in range(B_PER_W // L):
        i16 = idx_v[pl.ds(base + j * L, L)]
        lseg = plsc.load_gather(lse_v, [i16])
        elem = ebuf_v[pl.ds(j * L, L)]
        acc = acc + (lseg - elem)

    scatter_wait(g_end - 1, NCHUNK - 2, buf0, s0)
    scatter_wait(g_end - 1, NCHUNK - 1, buf1, s1)

    part_v[...] = acc
    pltpu.sync_copy(part_v, part_hbm.at[wid])


_sc_gather = functools.partial(
    pl.kernel,
    out_type=(
        jax.ShapeDtypeStruct((NG, NT0, 8, 128), jnp.float32),
        jax.ShapeDtypeStruct((NW, L), jnp.float32),
    ),
    mesh=plsc.VectorSubcoreMesh(
        core_axis_name="c", subcore_axis_name="s",
        num_cores=NC, num_subcores=NS),
    scratch_types=[
        pltpu.VMEM((N_ROWS,), jnp.int32),
        pltpu.VMEM((B_PER_W,), jnp.int32),
        pltpu.VMEM((V,), jnp.float32),
        pltpu.VMEM((32, V), jnp.float32),
        pltpu.VMEM((TCH, 8, 128), jnp.float32),
        pltpu.VMEM((TCH, 8, 128), jnp.float32),
        pltpu.VMEM((B_PER_W,), jnp.int32),
        pltpu.VMEM((B_PER_W,), jnp.float32),
        pltpu.VMEM((L,), jnp.float32),
        pltpu.SemaphoreType.DMA,
        pltpu.SemaphoreType.DMA,
        pltpu.SemaphoreType.DMA,
    ],
    compiler_params=pltpu.CompilerParams(
        use_tc_tiling_on_sc=False, needs_layout_passes=False),
)(_sc_body)


def kernel(idx, targets, table):
    idx_f = idx.reshape(-1).astype(jnp.int32)
    tgt_f = targets.reshape(-1).astype(jnp.int32)
    tabT = table.T
    tabTf = jnp.pad(tabT.reshape(V * V), (0, 8))
    lse = _row_lse(table)
    out4, parts = _sc_gather(tabT, tabTf, idx_f, tgt_f, lse)
    logits = out4.transpose(0, 2, 1, 3).reshape(V, N_ROWS).T
    loss = parts.sum() / jnp.float32(N_ROWS)
    return logits, loss


# speedup vs baseline: 1.1551x; 1.1551x over previous
import functools

import jax
import jax.numpy as jnp
from jax import lax
from jax.experimental import pallas as pl
from jax.experimental.pallas import tpu as pltpu
from jax.experimental.pallas import tpu_sc as plsc

NC = 2
NS = 16
NW = NC * NS
L = 16

V = 1000
N_ROWS = 51200
B_PER_W = N_ROWS // NW
NG = V // 8
NT0 = N_ROWS // 128
TCH = 8
NCHUNK = NT0 // TCH
NPAIR = NCHUNK // 2


def _lse_body(tab_ref, out_ref):
    x = tab_ref[...]
    m = jnp.max(x, axis=1, keepdims=True)
    s = jnp.sum(jnp.exp(x - m), axis=1, keepdims=True)
    out_ref[...] = m + jnp.log(s)


def _row_lse(table):
    out = pl.pallas_call(
        _lse_body,
        out_shape=jax.ShapeDtypeStruct((V, 1), jnp.float32),
    )(table)
    return out.reshape(V)


VF = V * V + 8


def _sc_body(tabT_hbm, tabTf_hbm, idx_hbm, tgt_hbm, lse_hbm,
             out_hbm, part_hbm,
             idx_v, tgt_v, lse_v, tabT_v, buf0, buf1,
             ilist_v, ebuf_v, part_v,
             s0, s1, esem):
    c = lax.axis_index("c")
    s = lax.axis_index("s")
    wid = s * NC + c
    base = wid * B_PER_W

    g_start = wid * NG // NW
    g_end = (wid + 1) * NG // NW

    pltpu.sync_copy(tabT_hbm.at[pl.ds(g_start * 8, 32)], tabT_v)
    pltpu.sync_copy(idx_hbm, idx_v)
    pltpu.sync_copy(tgt_hbm.at[pl.ds(base, B_PER_W)], tgt_v)
    pltpu.sync_copy(lse_hbm, lse_v)

    def fill(buf, g, chunk):
        rows = [tabT_v.at[(g - g_start) * 8 + c_loc] for c_loc in range(8)]

        @plsc.parallel_loop(0, TCH, step=1, unroll=2)
        def _(t):
            r0 = (chunk * TCH + t) * 128
            idx16 = [idx_v[pl.ds(r0 + jj * L, L)] for jj in range(8)]
            for c_loc in range(8):
                for jj in range(8):
                    buf[t, c_loc, pl.ds(jj * L, L)] = plsc.load_gather(
                        rows[c_loc], [idx16[jj]])

    def scatter_start(g, chunk, buf, sem):
        pltpu.async_copy(
            buf, out_hbm.at[g, pl.ds(chunk * TCH, TCH)], sem)

    def scatter_wait(g, chunk, buf, sem):
        pltpu.make_async_copy(
            buf, out_hbm.at[g, pl.ds(chunk * TCH, TCH)], sem).wait()

    for j in range(B_PER_W // L):
        i16 = idx_v[pl.ds(base + j * L, L)]
        t16 = tgt_v[pl.ds(j * L, L)]
        ilist_v[pl.ds(j * L, L)] = t16 * V + i16

    EG = 80
    for k in range(B_PER_W // EG):
        pltpu.async_copy(
            tabTf_hbm.at[ilist_v.at[pl.ds(k * EG, EG)]],
            ebuf_v.at[pl.ds(k * EG, EG)], esem)

    nq = (g_end - g_start) * NPAIR

    def q_body(q, carry):
        g = g_start + q // NPAIR
        p = q % NPAIR
        c0 = 2 * p
        c1 = 2 * p + 1

        @pl.when(q > 0)
        def _():
            scatter_wait(g, c0, buf0, s0)

        fill(buf0, g, c0)
        scatter_start(g, c0, buf0, s0)

        @pl.when(q > 0)
        def _():
            scatter_wait(g, c1, buf1, s1)

        fill(buf1, g, c1)
        scatter_start(g, c1, buf1, s1)
        return carry

    lax.fori_loop(0, nq, q_body, jnp.int32(0))

    for k in range(B_PER_W // EG):
        pltpu.make_async_copy(
            tabTf_hbm.at[ilist_v.at[pl.ds(k * EG, EG)]],
            ebuf_v.at[pl.ds(k * EG, EG)], esem).wait()

    acc = jnp.zeros((L,), jnp.float32)
    for j in range(B_PER_W // L):
        i16 = idx_v[pl.ds(base + j * L, L)]
        lseg = plsc.load_gather(lse_v, [i16])
        elem = ebuf_v[pl.ds(j * L, L)]
        acc = acc + (lseg - elem)

    scatter_wait(g_end - 1, NCHUNK - 2, buf0, s0)
    scatter_wait(g_end - 1, NCHUNK - 1, buf1, s1)

    part_v[...] = acc
    pltpu.sync_copy(part_v, part_hbm.at[wid])


_sc_gather = functools.partial(
    pl.kernel,
    out_type=(
        jax.ShapeDtypeStruct((NG, NT0, 8, 128), jnp.float32),
        jax.ShapeDtypeStruct((NW, L), jnp.float32),
    ),
    mesh=plsc.VectorSubcoreMesh(
        core_axis_name="c", subcore_axis_name="s",
        num_cores=NC, num_subcores=NS),
    scratch_types=[
        pltpu.VMEM((N_ROWS,), jnp.int32),
        pltpu.VMEM((B_PER_W,), jnp.int32),
        pltpu.VMEM((V,), jnp.float32),
        pltpu.VMEM((32, V), jnp.float32),
        pltpu.VMEM((TCH, 8, 128), jnp.float32),
        pltpu.VMEM((TCH, 8, 128), jnp.float32),
        pltpu.VMEM((B_PER_W,), jnp.int32),
        pltpu.VMEM((B_PER_W,), jnp.float32),
        pltpu.VMEM((L,), jnp.float32),
        pltpu.SemaphoreType.DMA,
        pltpu.SemaphoreType.DMA,
        pltpu.SemaphoreType.DMA,
    ],
    compiler_params=pltpu.CompilerParams(
        use_tc_tiling_on_sc=False, needs_layout_passes=False),
)(_sc_body)


def kernel(idx, targets, table):
    idx_f = idx.reshape(-1).astype(jnp.int32)
    tgt_f = targets.reshape(-1).astype(jnp.int32)
    tabT = table.T
    tabTf = jnp.pad(tabT.reshape(V * V), (0, 8))
    lse = _row_lse(table)
    out4, parts = _sc_gather(tabT, tabTf, idx_f, tgt_f, lse)
    logits = out4.transpose(0, 2, 1, 3).reshape(V, N_ROWS).T
    loss = parts.sum() / jnp.float32(N_ROWS)
    return logits, loss
